# Initial kernel scaffold; baseline (speedup 1.0000x reference)
import jax
import jax.numpy as jnp
from jax.experimental import pallas as pl


def kernel(output, target, segment_ids):
    raise NotImplementedError("write your pallas kernel here")


# speedup vs baseline: 3.5703x; 3.5703x over previous
import functools

import jax
import jax.numpy as jnp
from jax import lax
from jax.experimental import pallas as pl
from jax.experimental.pallas import tpu as pltpu
from jax.experimental.pallas import tpu_sc as plsc

N = 1048576
C = 64
NSEG = 1024
HIST = NSEG * C
NC, NS, L = 2, 16, 16
NW = NC * NS
EPW = N // NW
CH = 2048
NCHUNK = EPW // CH
NB = 1024
RB = N // NB

@functools.cache
def _mesh():
    return plsc.VectorSubcoreMesh(
        core_axis_name="c", subcore_axis_name="s",
        num_cores=NC, num_subcores=NS)


def _iota16():
    return lax.iota(jnp.int32, L)


def _hist_body(seg_hbm, tgt_hbm, out_hbm, seg_v, tgt_v, idx_rows, ones_v,
               zero_v, hist_sp, sem):
    cid = lax.axis_index("c")
    sid = lax.axis_index("s")
    wid = cid * NS + sid

    ones = jnp.ones((L,), jnp.int32)
    zeros = jnp.zeros((L,), jnp.int32)
    for k in range(128 // L):
        ones_v[pl.ds(k * L, L)] = ones

    def _zero_body(i, _):
        zero_v[pl.ds(i * L, L)] = zeros
        return 0
    lax.fori_loop(0, (HIST // NS) // L, _zero_body, 0)
    pltpu.sync_copy(zero_v, hist_sp.at[pl.ds(sid * (HIST // NS), HIST // NS)])
    plsc.subcore_barrier()

    def _chunk(g, _):
        base = wid * EPW + g * CH
        pltpu.sync_copy(seg_hbm.at[pl.ds(base, CH)], seg_v)
        pltpu.sync_copy(tgt_hbm.at[pl.ds(base, CH)], tgt_v)
        for j in range(CH // 128):
            for k in range(128 // L):
                off = j * 128 + k * L
                s = seg_v[pl.ds(off, L)]
                t = tgt_v[pl.ds(off, L)]
                idx_rows[j][pl.ds(k * L, L)] = s * C + t
        descs = [
            pltpu.async_copy(ones_v, hist_sp.at[idx_rows[j]], sem, add=True)
            for j in range(CH // 128)
        ]
        for d in descs:
            d.wait()
        return 0
    lax.fori_loop(0, NCHUNK, _chunk, 0)

    plsc.subcore_barrier()
    sl = HIST // NS
    pltpu.sync_copy(hist_sp.at[pl.ds(sid * sl, sl)],
                    out_hbm.at[pl.ds(cid * HIST + sid * sl, sl)])


def _argmax_body(hists_hbm, out_hbm, h0, h1, res, sem):
    cid = lax.axis_index("c")
    sid = lax.axis_index("s")
    wid = cid * NS + sid
    spw = NSEG // NW
    nel = spw * C
    pltpu.sync_copy(hists_hbm.at[pl.ds(wid * nel, nel)], h0)
    pltpu.sync_copy(hists_hbm.at[pl.ds(HIST + wid * nel, nel)], h1)
    iota = _iota16()
    for g in range(spw // L):
        best = jnp.full((L,), -1, jnp.int32)
        besti = jnp.zeros((L,), jnp.int32)
        for c in range(C):
            idx = iota * C + (g * L * C + c)
            cnt = plsc.load_gather(h0, [idx]) + plsc.load_gather(h1, [idx])
            upd = cnt > best
            best = jnp.where(upd, cnt, best)
            besti = jnp.where(upd, jnp.full((L,), c, jnp.int32), besti)
        res[pl.ds(g * L, L)] = besti
    pltpu.sync_copy(res, out_hbm.at[pl.ds(wid * spw, spw)])


def _pick_body(outflat_hbm, seg_hbm, maj_hbm, out_hbm, maj_v, seg_v, idx_v,
               val_v, acc_v, sem):
    cid = lax.axis_index("c")
    sid = lax.axis_index("s")
    wid = cid * NS + sid
    pltpu.sync_copy(maj_hbm, maj_v)
    acc_v[...] = jnp.zeros((L,), jnp.float32)
    iota = _iota16()

    def _chunk(g, _):
        base = wid * EPW + g * CH
        pltpu.sync_copy(seg_hbm.at[pl.ds(base, CH)], seg_v)
        for j in range(CH // 128):
            for k in range(128 // L):
                off = j * 128 + k * L
                s = seg_v[pl.ds(off, L)]
                m = plsc.load_gather(maj_v, [s])
                idx_v[pl.ds(off, L)] = (base + off + iota) * C + m
        descs = [
            pltpu.async_copy(outflat_hbm.at[idx_v.at[pl.ds(j * 128, 128)]],
                             val_v.at[pl.ds(j * 128, 128)], sem)
            for j in range(CH // 128)
        ]
        for d in descs:
            d.wait()
        acc = acc_v[...]
        for j in range(CH // L):
            acc = acc + val_v[pl.ds(j * L, L)]
        acc_v[...] = acc
        return 0
    lax.fori_loop(0, NCHUNK, _chunk, 0)

    pltpu.sync_copy(acc_v, out_hbm.at[pl.ds(wid * L, L)])
    acc_v[...] = jnp.zeros((L,), jnp.float32)
    pltpu.sync_copy(acc_v, out_hbm.at[pl.ds(NW * L + wid * L, L)])


def _lse_body(x_ref, part_ref, out_ref, acc_ref):
    i = pl.program_id(0)

    @pl.when(i == 0)
    def _():
        acc_ref[0] = 0.0

    xb = x_ref[0]
    s = jnp.sum(jnp.exp(xb), axis=-1)
    acc_ref[0] += jnp.sum(jnp.log(s))

    @pl.when(i == NB - 1)
    def _():
        val = (acc_ref[0] - jnp.sum(part_ref[...])) / N
        out_ref[...] = jnp.broadcast_to(val, (1, 1))


@functools.cache
def _sc_kernels():
    sc_params = pltpu.CompilerParams(needs_layout_passes=False)
    hist_k = pl.kernel(
        _hist_body,
        out_type=jax.ShapeDtypeStruct((NC * HIST,), jnp.int32),
        mesh=_mesh(),
        scratch_types=[
            pltpu.VMEM((CH,), jnp.int32),
            pltpu.VMEM((CH,), jnp.int32),
            [pltpu.VMEM((128,), jnp.int32) for _ in range(CH // 128)],
            pltpu.VMEM((128,), jnp.int32),
            pltpu.VMEM((HIST // NS,), jnp.int32),
            pltpu.VMEM_SHARED((HIST,), jnp.int32),
            pltpu.SemaphoreType.DMA,
        ],
        compiler_params=sc_params,
    )
    argmax_k = pl.kernel(
        _argmax_body,
        out_type=jax.ShapeDtypeStruct((NSEG,), jnp.int32),
        mesh=_mesh(),
        scratch_types=[
            pltpu.VMEM((NSEG // NW * C,), jnp.int32),
            pltpu.VMEM((NSEG // NW * C,), jnp.int32),
            pltpu.VMEM((NSEG // NW,), jnp.int32),
            pltpu.SemaphoreType.DMA,
        ],
        compiler_params=sc_params,
    )
    pick_k = pl.kernel(
        _pick_body,
        out_type=jax.ShapeDtypeStruct((NW * L * 2,), jnp.float32),
        mesh=_mesh(),
        scratch_types=[
            pltpu.VMEM((NSEG,), jnp.int32),
            pltpu.VMEM((CH,), jnp.int32),
            pltpu.VMEM((CH,), jnp.int32),
            pltpu.VMEM((CH,), jnp.float32),
            pltpu.VMEM((L,), jnp.float32),
            pltpu.SemaphoreType.DMA,
        ],
        compiler_params=sc_params,
    )
    return hist_k, argmax_k, pick_k


def kernel(output, target, segment_ids):
    hist_k, argmax_k, pick_k = _sc_kernels()
    seg = segment_ids.astype(jnp.int32)
    tgt = target.astype(jnp.int32)
    hists = hist_k(seg, tgt)
    majority = argmax_k(hists)
    partials = pick_k(output.reshape(-1), seg, majority)
    loss = pl.pallas_call(
        _lse_body,
        grid=(NB,),
        in_specs=[
            pl.BlockSpec((1, RB, C), lambda i: (i, 0, 0)),
            pl.BlockSpec((8, 128), lambda i: (0, 0)),
        ],
        out_specs=pl.BlockSpec((1, 1), lambda i: (0, 0)),
        out_shape=jax.ShapeDtypeStruct((1, 1), jnp.float32),
        scratch_shapes=[pltpu.SMEM((1,), jnp.float32)],
    )(output.reshape(NB, RB, C), partials.reshape(8, 128))
    return loss.reshape(())


# speedup vs baseline: 20.5955x; 5.7686x over previous
import functools

import jax
import jax.numpy as jnp
from jax import lax
from jax.experimental import pallas as pl
from jax.experimental.pallas import tpu as pltpu
from jax.experimental.pallas import tpu_sc as plsc

N = 1048576
C = 64
NSEG = 1024
HIST = NSEG * C
NC, NS, L = 2, 16, 16
NW = NC * NS
EPW = N // NW
CH = 2048
NCHUNK = EPW // CH
NB = 1024
RB = N // NB

@functools.cache
def _mesh():
    return plsc.VectorSubcoreMesh(
        core_axis_name="c", subcore_axis_name="s",
        num_cores=NC, num_subcores=NS)


def _iota16():
    return lax.iota(jnp.int32, L)


def _hist_body(seg_hbm, tgt_hbm, out_hbm, seg_v, tgt_v, idx_rows, ones_v,
               zero_v, hist_sp, sem):
    cid = lax.axis_index("c")
    sid = lax.axis_index("s")
    wid = cid * NS + sid

    ones = jnp.ones((L,), jnp.int32)
    zeros = jnp.zeros((L,), jnp.int32)
    for k in range(128 // L):
        ones_v[pl.ds(k * L, L)] = ones

    def _zero_body(i, _):
        zero_v[pl.ds(i * L, L)] = zeros
        return 0
    lax.fori_loop(0, (HIST // NS) // L, _zero_body, 0)
    pltpu.sync_copy(zero_v, hist_sp.at[pl.ds(sid * (HIST // NS), HIST // NS)])
    plsc.subcore_barrier()

    def _chunk(g, _):
        base = wid * EPW + g * CH
        pltpu.sync_copy(seg_hbm.at[pl.ds(base, CH)], seg_v)
        pltpu.sync_copy(tgt_hbm.at[pl.ds(base, CH)], tgt_v)
        for j in range(CH // 128):
            for k in range(128 // L):
                off = j * 128 + k * L
                s = seg_v[pl.ds(off, L)]
                t = tgt_v[pl.ds(off, L)]
                idx_rows[j][pl.ds(k * L, L)] = s * C + t
        descs = [
            pltpu.async_copy(ones_v, hist_sp.at[idx_rows[j]], sem, add=True)
            for j in range(CH // 128)
        ]
        for d in descs:
            d.wait()
        return 0
    lax.fori_loop(0, NCHUNK, _chunk, 0)

    plsc.subcore_barrier()
    sl = HIST // NS
    pltpu.sync_copy(hist_sp.at[pl.ds(sid * sl, sl)],
                    out_hbm.at[pl.ds(cid * HIST + sid * sl, sl)])


def _argmax_body(hists_hbm, out_hbm, h0, h1, res, sem):
    cid = lax.axis_index("c")
    sid = lax.axis_index("s")
    wid = cid * NS + sid
    spw = NSEG // NW
    nel = spw * C
    pltpu.sync_copy(hists_hbm.at[pl.ds(wid * nel, nel)], h0)
    pltpu.sync_copy(hists_hbm.at[pl.ds(HIST + wid * nel, nel)], h1)
    iota = _iota16()
    for g in range(spw // L):
        best = jnp.full((L,), -1, jnp.int32)
        besti = jnp.zeros((L,), jnp.int32)
        for c in range(C):
            idx = iota * C + (g * L * C + c)
            cnt = plsc.load_gather(h0, [idx]) + plsc.load_gather(h1, [idx])
            upd = cnt > best
            best = jnp.where(upd, cnt, best)
            besti = jnp.where(upd, jnp.full((L,), c, jnp.int32), besti)
        res[pl.ds(g * L, L)] = besti
    pltpu.sync_copy(res, out_hbm.at[pl.ds(wid * spw, spw)])


def _pick_body(outflat_hbm, seg_hbm, maj_hbm, out_hbm, maj_v, seg_v, idx_v,
               val_v, acc_v, sem):
    cid = lax.axis_index("c")
    sid = lax.axis_index("s")
    wid = cid * NS + sid
    pltpu.sync_copy(maj_hbm, maj_v)
    acc_v[...] = jnp.zeros((L,), jnp.float32)
    iota = _iota16()

    def _chunk(g, _):
        base = wid * EPW + g * CH
        pltpu.sync_copy(seg_hbm.at[pl.ds(base, CH)], seg_v)
        for j in range(CH // 128):
            for k in range(128 // L):
                off = j * 128 + k * L
                s = seg_v[pl.ds(off, L)]
                m = plsc.load_gather(maj_v, [s])
                i_ = base + off + iota
                idx_v[pl.ds(off, L)] = (
                    (m >> 3) * (N * 8) + (i_ >> 7) * 1024
                    + (m & 7) * 128 + (i_ & 127))
        descs = [
            pltpu.async_copy(outflat_hbm.at[idx_v.at[pl.ds(j * 128, 128)]],
                             val_v.at[pl.ds(j * 128, 128)], sem)
            for j in range(CH // 128)
        ]
        for d in descs:
            d.wait()
        acc = acc_v[...]
        for j in range(CH // L):
            acc = acc + val_v[pl.ds(j * L, L)]
        acc_v[...] = acc
        return 0
    lax.fori_loop(0, NCHUNK, _chunk, 0)

    pltpu.sync_copy(acc_v, out_hbm.at[pl.ds(wid * L, L)])
    acc_v[...] = jnp.zeros((L,), jnp.float32)
    pltpu.sync_copy(acc_v, out_hbm.at[pl.ds(NW * L + wid * L, L)])


BLKC = 8192
NBT = N // BLKC


def _lse_body(x_ref, out_ref, acc_ref):
    i = pl.program_id(0)

    @pl.when(i == 0)
    def _():
        acc_ref[0] = 0.0

    xb = x_ref[...]
    s = jnp.sum(jnp.exp(xb), axis=0)
    acc_ref[0] += jnp.sum(jnp.log(s))

    @pl.when(i == NBT - 1)
    def _():
        out_ref[...] = jnp.broadcast_to(acc_ref[0], (1, 1))


def _combine_body(lse_ref, part_ref, out_ref):
    val = (lse_ref[0, 0] - jnp.sum(part_ref[...])) / N
    out_ref[...] = jnp.broadcast_to(val, (1, 1))


@functools.cache
def _sc_kernels():
    sc_params = pltpu.CompilerParams(needs_layout_passes=False)
    hist_k = pl.kernel(
        _hist_body,
        out_type=jax.ShapeDtypeStruct((NC * HIST,), jnp.int32),
        mesh=_mesh(),
        scratch_types=[
            pltpu.VMEM((CH,), jnp.int32),
            pltpu.VMEM((CH,), jnp.int32),
            [pltpu.VMEM((128,), jnp.int32) for _ in range(CH // 128)],
            pltpu.VMEM((128,), jnp.int32),
            pltpu.VMEM((HIST // NS,), jnp.int32),
            pltpu.VMEM_SHARED((HIST,), jnp.int32),
            pltpu.SemaphoreType.DMA,
        ],
        compiler_params=sc_params,
    )
    argmax_k = pl.kernel(
        _argmax_body,
        out_type=jax.ShapeDtypeStruct((NSEG,), jnp.int32),
        mesh=_mesh(),
        scratch_types=[
            pltpu.VMEM((NSEG // NW * C,), jnp.int32),
            pltpu.VMEM((NSEG // NW * C,), jnp.int32),
            pltpu.VMEM((NSEG // NW,), jnp.int32),
            pltpu.SemaphoreType.DMA,
        ],
        compiler_params=sc_params,
    )
    pick_k = pl.kernel(
        _pick_body,
        out_type=jax.ShapeDtypeStruct((NW * L * 2,), jnp.float32),
        mesh=_mesh(),
        scratch_types=[
            pltpu.VMEM((NSEG,), jnp.int32),
            pltpu.VMEM((CH,), jnp.int32),
            pltpu.VMEM((CH,), jnp.int32),
            pltpu.VMEM((CH,), jnp.float32),
            pltpu.VMEM((L,), jnp.float32),
            pltpu.SemaphoreType.DMA,
        ],
        compiler_params=sc_params,
    )
    return hist_k, argmax_k, pick_k


def kernel(output, target, segment_ids):
    hist_k, argmax_k, pick_k = _sc_kernels()
    seg = segment_ids.astype(jnp.int32)
    tgt = target.astype(jnp.int32)
    hists = hist_k(seg, tgt)
    majority = argmax_k(hists)
    xt = output.T
    flat = (xt.reshape(8, 8, N // 128, 128)
              .transpose(0, 2, 1, 3).reshape(-1))
    partials = pick_k(flat, seg, majority)
    sum_lse = pl.pallas_call(
        _lse_body,
        grid=(NBT,),
        in_specs=[pl.BlockSpec((C, BLKC), lambda i: (0, i))],
        out_specs=pl.BlockSpec((1, 1), lambda i: (0, 0)),
        out_shape=jax.ShapeDtypeStruct((1, 1), jnp.float32),
        scratch_shapes=[pltpu.SMEM((1,), jnp.float32)],
    )(xt)
    loss = pl.pallas_call(
        _combine_body,
        in_specs=[
            pl.BlockSpec((1, 1), lambda: (0, 0)),
            pl.BlockSpec((8, 128), lambda: (0, 0)),
        ],
        out_specs=pl.BlockSpec((1, 1), lambda: (0, 0)),
        out_shape=jax.ShapeDtypeStruct((1, 1), jnp.float32),
    )(sum_lse, partials.reshape(8, 128))
    return loss.reshape(())


# speedup vs baseline: 25.1600x; 1.2216x over previous
import functools

import jax
import jax.numpy as jnp
from jax import lax
from jax.experimental import pallas as pl
from jax.experimental.pallas import tpu as pltpu
from jax.experimental.pallas import tpu_sc as plsc

N = 1048576
C = 64
NSEG = 1024
HIST = NSEG * C
NC, NS, L = 2, 16, 16
NW = NC * NS
EPW = N // NW
CH = 2048
NCHUNK = EPW // CH
NB = 1024
RB = N // NB

@functools.cache
def _mesh():
    return plsc.VectorSubcoreMesh(
        core_axis_name="c", subcore_axis_name="s",
        num_cores=NC, num_subcores=NS)


def _iota16():
    return lax.iota(jnp.int32, L)


def _hist_body(seg_hbm, tgt_hbm, out_hbm, seg_v, tgt_v, idx_rows, ones_v,
               zero_v, hist_sp, sem, sems):
    cid = lax.axis_index("c")
    sid = lax.axis_index("s")
    wid = cid * NS + sid

    ones = jnp.ones((L,), jnp.int32)
    zeros = jnp.zeros((L,), jnp.int32)
    for k in range(128 // L):
        ones_v[pl.ds(k * L, L)] = ones

    def _zero_body(i, _):
        zero_v[pl.ds(i * L, L)] = zeros
        return 0
    lax.fori_loop(0, (HIST // NS) // L, _zero_body, 0)
    pltpu.sync_copy(zero_v, hist_sp.at[pl.ds(sid * (HIST // NS), HIST // NS)])
    plsc.subcore_barrier()

    def _start_stage(g, slot):
        base = wid * EPW + g * CH
        pltpu.async_copy(seg_hbm.at[pl.ds(base, CH)], seg_v[slot], sems[slot])
        pltpu.async_copy(tgt_hbm.at[pl.ds(base, CH)], tgt_v[slot], sems[slot])

    _start_stage(0, 0)

    def _pair(g2, _):
        for b in range(2):
            g = 2 * g2 + b

            @pl.when(g + 1 < NCHUNK)
            def _():
                _start_stage(g + 1, 1 - b)

            pltpu.make_async_copy(
                seg_hbm.at[pl.ds(0, CH)], seg_v[b], sems[b]).wait()
            pltpu.make_async_copy(
                tgt_hbm.at[pl.ds(0, CH)], tgt_v[b], sems[b]).wait()
            for j in range(CH // 128):
                for k in range(128 // L):
                    off = j * 128 + k * L
                    s = seg_v[b][pl.ds(off, L)]
                    t = tgt_v[b][pl.ds(off, L)]
                    idx_rows[j][pl.ds(k * L, L)] = s * C + t
            descs = [
                pltpu.async_copy(ones_v, hist_sp.at[idx_rows[j]], sem,
                                 add=True)
                for j in range(CH // 128)
            ]
            for d in descs:
                d.wait()
        return 0
    lax.fori_loop(0, NCHUNK // 2, _pair, 0)

    plsc.subcore_barrier()
    sl = HIST // NS
    pltpu.sync_copy(hist_sp.at[pl.ds(sid * sl, sl)],
                    out_hbm.at[pl.ds(cid * HIST + sid * sl, sl)])


def _argmax_body(hists_hbm, out_hbm, h0, h1, res, sem):
    cid = lax.axis_index("c")
    sid = lax.axis_index("s")
    wid = cid * NS + sid
    spw = NSEG // NW
    nel = spw * C
    pltpu.sync_copy(hists_hbm.at[pl.ds(wid * nel, nel)], h0)
    pltpu.sync_copy(hists_hbm.at[pl.ds(HIST + wid * nel, nel)], h1)
    iota = _iota16()
    for g in range(spw // L):
        best = jnp.full((L,), -1, jnp.int32)
        besti = jnp.zeros((L,), jnp.int32)
        for c in range(C):
            idx = iota * C + (g * L * C + c)
            cnt = plsc.load_gather(h0, [idx]) + plsc.load_gather(h1, [idx])
            upd = cnt > best
            best = jnp.where(upd, cnt, best)
            besti = jnp.where(upd, jnp.full((L,), c, jnp.int32), besti)
        res[pl.ds(g * L, L)] = besti
    pltpu.sync_copy(res, out_hbm.at[pl.ds(wid * spw, spw)])


def _pick_body(outflat_hbm, seg_hbm, maj_hbm, out_hbm, maj_v, seg_v, idx_v,
               val_v, acc_v, sem, sems):
    cid = lax.axis_index("c")
    sid = lax.axis_index("s")
    wid = cid * NS + sid
    pltpu.sync_copy(maj_hbm, maj_v)
    acc_v[...] = jnp.zeros((L,), jnp.float32)
    iota = _iota16()

    def _start_stage(g, slot):
        base = wid * EPW + g * CH
        pltpu.async_copy(seg_hbm.at[pl.ds(base, CH)], seg_v[slot], sems[slot])

    _start_stage(0, 0)

    def _pair(g2, _):
        for b in range(2):
            g = 2 * g2 + b
            base = wid * EPW + g * CH

            @pl.when(g + 1 < NCHUNK)
            def _():
                _start_stage(g + 1, 1 - b)

            pltpu.make_async_copy(
                seg_hbm.at[pl.ds(0, CH)], seg_v[b], sems[b]).wait()
            for j in range(CH // 128):
                off = j * 128
                i0 = base + off
                first = seg_v[b][pl.ds(off, L)]
                last = seg_v[b][pl.ds(off + 128 - L, L)]
                s0 = jnp.min(first)
                s1 = jnp.max(last)

                def _uniform(off=off, i0=i0, first=first):
                    m = jnp.max(plsc.load_gather(maj_v, [first]))
                    bw = (m >> 3) * (N * 8) + (i0 >> 7) * 1024 + (m & 7) * 128
                    pltpu.async_copy(outflat_hbm.at[pl.ds(bw, 128)],
                                     val_v.at[pl.ds(off, 128)], sem)

                def _mixed(off=off, i0=i0):
                    for k in range(128 // L):
                        o2 = off + k * L
                        s = seg_v[b][pl.ds(o2, L)]
                        m = plsc.load_gather(maj_v, [s])
                        i_ = i0 + k * L + iota
                        idx_v[pl.ds(o2, L)] = (
                            (m >> 3) * (N * 8) + (i_ >> 7) * 1024
                            + (m & 7) * 128 + (i_ & 127))
                    pltpu.async_copy(
                        outflat_hbm.at[idx_v.at[pl.ds(off, 128)]],
                        val_v.at[pl.ds(off, 128)], sem)

                lax.cond(s0 == s1, _uniform, _mixed)
            for j in range(CH // 128):
                pltpu.make_async_copy(
                    outflat_hbm.at[pl.ds(0, 128)],
                    val_v.at[pl.ds(j * 128, 128)], sem).wait()
            acc = acc_v[...]
            for j in range(CH // L):
                acc = acc + val_v[pl.ds(j * L, L)]
            acc_v[...] = acc
        return 0
    lax.fori_loop(0, NCHUNK // 2, _pair, 0)

    pltpu.sync_copy(acc_v, out_hbm.at[pl.ds(wid * L, L)])
    acc_v[...] = jnp.zeros((L,), jnp.float32)
    pltpu.sync_copy(acc_v, out_hbm.at[pl.ds(NW * L + wid * L, L)])


BLKC = 8192
NBT = N // BLKC


def _lse_body(x_ref, out_ref, acc_ref):
    i = pl.program_id(0)

    @pl.when(i == 0)
    def _():
        acc_ref[0] = 0.0

    xb = x_ref[...]
    s = jnp.sum(jnp.exp(xb), axis=0)
    acc_ref[0] += jnp.sum(jnp.log(s))

    @pl.when(i == NBT - 1)
    def _():
        out_ref[...] = jnp.broadcast_to(acc_ref[0], (1, 1))


def _combine_body(lse_ref, part_ref, out_ref):
    val = (lse_ref[0, 0] - jnp.sum(part_ref[...])) / N
    out_ref[...] = jnp.broadcast_to(val, (1, 1))


@functools.cache
def _sc_kernels():
    sc_params = pltpu.CompilerParams(needs_layout_passes=False)
    hist_k = pl.kernel(
        _hist_body,
        out_type=jax.ShapeDtypeStruct((NC * HIST,), jnp.int32),
        mesh=_mesh(),
        scratch_types=[
            [pltpu.VMEM((CH,), jnp.int32) for _ in range(2)],
            [pltpu.VMEM((CH,), jnp.int32) for _ in range(2)],
            [pltpu.VMEM((128,), jnp.int32) for _ in range(CH // 128)],
            pltpu.VMEM((128,), jnp.int32),
            pltpu.VMEM((HIST // NS,), jnp.int32),
            pltpu.VMEM_SHARED((HIST,), jnp.int32),
            pltpu.SemaphoreType.DMA,
            [pltpu.SemaphoreType.DMA for _ in range(2)],
        ],
        compiler_params=sc_params,
    )
    argmax_k = pl.kernel(
        _argmax_body,
        out_type=jax.ShapeDtypeStruct((NSEG,), jnp.int32),
        mesh=_mesh(),
        scratch_types=[
            pltpu.VMEM((NSEG // NW * C,), jnp.int32),
            pltpu.VMEM((NSEG // NW * C,), jnp.int32),
            pltpu.VMEM((NSEG // NW,), jnp.int32),
            pltpu.SemaphoreType.DMA,
        ],
        compiler_params=sc_params,
    )
    pick_k = pl.kernel(
        _pick_body,
        out_type=jax.ShapeDtypeStruct((NW * L * 2,), jnp.float32),
        mesh=_mesh(),
        scratch_types=[
            pltpu.VMEM((NSEG,), jnp.int32),
            [pltpu.VMEM((CH,), jnp.int32) for _ in range(2)],
            pltpu.VMEM((CH,), jnp.int32),
            pltpu.VMEM((CH,), jnp.float32),
            pltpu.VMEM((L,), jnp.float32),
            pltpu.SemaphoreType.DMA,
            [pltpu.SemaphoreType.DMA for _ in range(2)],
        ],
        compiler_params=sc_params,
    )
    return hist_k, argmax_k, pick_k


def kernel(output, target, segment_ids):
    hist_k, argmax_k, pick_k = _sc_kernels()
    seg = segment_ids.astype(jnp.int32)
    tgt = target.astype(jnp.int32)
    hists = hist_k(seg, tgt)
    majority = argmax_k(hists)
    xt = output.T
    flat = (xt.reshape(8, 8, N // 128, 128)
              .transpose(0, 2, 1, 3).reshape(-1))
    partials = pick_k(flat, seg, majority)
    sum_lse = pl.pallas_call(
        _lse_body,
        grid=(NBT,),
        in_specs=[pl.BlockSpec((C, BLKC), lambda i: (0, i))],
        out_specs=pl.BlockSpec((1, 1), lambda i: (0, 0)),
        out_shape=jax.ShapeDtypeStruct((1, 1), jnp.float32),
        scratch_shapes=[pltpu.SMEM((1,), jnp.float32)],
    )(xt)
    loss = pl.pallas_call(
        _combine_body,
        in_specs=[
            pl.BlockSpec((1, 1), lambda: (0, 0)),
            pl.BlockSpec((8, 128), lambda: (0, 0)),
        ],
        out_specs=pl.BlockSpec((1, 1), lambda: (0, 0)),
        out_shape=jax.ShapeDtypeStruct((1, 1), jnp.float32),
    )(sum_lse, partials.reshape(8, 128))
    return loss.reshape(())


# speedup vs baseline: 25.1863x; 1.0010x over previous
import functools

import jax
import jax.numpy as jnp
from jax import lax
from jax.experimental import pallas as pl
from jax.experimental.pallas import tpu as pltpu
from jax.experimental.pallas import tpu_sc as plsc

N = 1048576
C = 64
NSEG = 1024
HIST = NSEG * C
NC, NS, L = 2, 16, 16
NW = NC * NS
EPW = N // NW
CH = 2048
NCHUNK = EPW // CH
NB = 1024
RB = N // NB

@functools.cache
def _mesh():
    return plsc.VectorSubcoreMesh(
        core_axis_name="c", subcore_axis_name="s",
        num_cores=NC, num_subcores=NS)


def _iota16():
    return lax.iota(jnp.int32, L)


def _hist_body(seg_hbm, tgt_hbm, out_hbm, seg_v, tgt_v, idx_rows, ones_v,
               zero_v, hist_sp, sem, sems):
    cid = lax.axis_index("c")
    sid = lax.axis_index("s")
    wid = cid * NS + sid

    ones = jnp.ones((L,), jnp.int32)
    zeros = jnp.zeros((L,), jnp.int32)
    for k in range(128 // L):
        ones_v[pl.ds(k * L, L)] = ones

    def _zero_body(i, _):
        zero_v[pl.ds(i * L, L)] = zeros
        return 0
    lax.fori_loop(0, (HIST // NS) // L, _zero_body, 0)
    pltpu.sync_copy(zero_v, hist_sp.at[pl.ds(sid * (HIST // NS), HIST // NS)])
    plsc.subcore_barrier()

    def _start_stage(g, slot):
        base = wid * EPW + g * CH
        pltpu.async_copy(seg_hbm.at[pl.ds(base, CH)], seg_v[slot], sems[slot])
        pltpu.async_copy(tgt_hbm.at[pl.ds(base, CH)], tgt_v[slot], sems[slot])

    _start_stage(0, 0)

    def _pair(g2, _):
        for b in range(2):
            g = 2 * g2 + b

            @pl.when(g + 1 < NCHUNK)
            def _():
                _start_stage(g + 1, 1 - b)

            pltpu.make_async_copy(
                seg_hbm.at[pl.ds(0, CH)], seg_v[b], sems[b]).wait()
            pltpu.make_async_copy(
                tgt_hbm.at[pl.ds(0, CH)], tgt_v[b], sems[b]).wait()
            for j in range(CH // 128):
                for k in range(128 // L):
                    off = j * 128 + k * L
                    s = seg_v[b][pl.ds(off, L)]
                    t = tgt_v[b][pl.ds(off, L)]
                    idx_rows[j][pl.ds(k * L, L)] = s * C + t
            descs = [
                pltpu.async_copy(ones_v, hist_sp.at[idx_rows[j]], sem,
                                 add=True)
                for j in range(CH // 128)
            ]
            for d in descs:
                d.wait()
        return 0
    lax.fori_loop(0, NCHUNK // 2, _pair, 0)

    plsc.subcore_barrier()
    sl = HIST // NS
    pltpu.sync_copy(hist_sp.at[pl.ds(sid * sl, sl)],
                    out_hbm.at[pl.ds(cid * HIST + sid * sl, sl)])


def _argmax_body(hists_hbm, out_hbm, h0, h1, res, sem):
    cid = lax.axis_index("c")
    sid = lax.axis_index("s")
    wid = cid * NS + sid
    spw = NSEG // NW
    nel = spw * C
    pltpu.sync_copy(hists_hbm.at[pl.ds(wid * nel, nel)], h0)
    pltpu.sync_copy(hists_hbm.at[pl.ds(HIST + wid * nel, nel)], h1)
    iota = _iota16()
    for g in range(spw // L):
        best = jnp.full((L,), -1, jnp.int32)
        besti = jnp.zeros((L,), jnp.int32)
        for c in range(C):
            idx = iota * C + (g * L * C + c)
            cnt = plsc.load_gather(h0, [idx]) + plsc.load_gather(h1, [idx])
            upd = cnt > best
            best = jnp.where(upd, cnt, best)
            besti = jnp.where(upd, jnp.full((L,), c, jnp.int32), besti)
        res[pl.ds(g * L, L)] = besti
    pltpu.sync_copy(res, out_hbm.at[pl.ds(wid * spw, spw)])


def _pick_body(outflat_hbm, seg_hbm, maj_hbm, out_hbm, maj_v, seg_v, idx_v,
               val_v, acc_v, sem, sems):
    cid = lax.axis_index("c")
    sid = lax.axis_index("s")
    wid = cid * NS + sid
    pltpu.sync_copy(maj_hbm, maj_v)
    acc_v[...] = jnp.zeros((L,), jnp.float32)
    iota = _iota16()

    def _start_stage(g, slot):
        base = wid * EPW + g * CH
        pltpu.async_copy(seg_hbm.at[pl.ds(base, CH)], seg_v[slot], sems[slot])

    _start_stage(0, 0)

    def _pair(g2, _):
        for b in range(2):
            g = 2 * g2 + b
            base = wid * EPW + g * CH

            @pl.when(g + 1 < NCHUNK)
            def _():
                _start_stage(g + 1, 1 - b)

            pltpu.make_async_copy(
                seg_hbm.at[pl.ds(0, CH)], seg_v[b], sems[b]).wait()
            for j in range(CH // 128):
                off = j * 128
                i0 = base + off
                first = seg_v[b][pl.ds(off, L)]
                last = seg_v[b][pl.ds(off + 128 - L, L)]
                s0 = jnp.min(first)
                s1 = jnp.max(last)

                def _uniform(off=off, i0=i0, first=first):
                    m = jnp.max(plsc.load_gather(maj_v, [first]))
                    bw = (m >> 3) * (N * 8) + (i0 >> 7) * 1024 + (m & 7) * 128
                    pltpu.async_copy(outflat_hbm.at[pl.ds(bw, 128)],
                                     val_v.at[pl.ds(off, 128)], sem)

                def _mixed(off=off, i0=i0):
                    for k in range(128 // L):
                        o2 = off + k * L
                        s = seg_v[b][pl.ds(o2, L)]
                        m = plsc.load_gather(maj_v, [s])
                        i_ = i0 + k * L + iota
                        idx_v[pl.ds(o2, L)] = (
                            (m >> 3) * (N * 8) + (i_ >> 7) * 1024
                            + (m & 7) * 128 + (i_ & 127))
                    pltpu.async_copy(
                        outflat_hbm.at[idx_v.at[pl.ds(off, 128)]],
                        val_v.at[pl.ds(off, 128)], sem)

                lax.cond(s0 == s1, _uniform, _mixed)
            for j in range(CH // 128):
                pltpu.make_async_copy(
                    outflat_hbm.at[pl.ds(0, 128)],
                    val_v.at[pl.ds(j * 128, 128)], sem).wait()
            acc = acc_v[...]
            for j in range(CH // L):
                acc = acc + val_v[pl.ds(j * L, L)]
            acc_v[...] = acc
        return 0
    lax.fori_loop(0, NCHUNK // 2, _pair, 0)

    pltpu.sync_copy(acc_v, out_hbm.at[pl.ds(wid * L, L)])
    acc_v[...] = jnp.zeros((L,), jnp.float32)
    pltpu.sync_copy(acc_v, out_hbm.at[pl.ds(NW * L + wid * L, L)])


BLKC = 8192
NBT = N // BLKC


def _lse_body(x_ref, out_ref, acc_ref):
    i = pl.program_id(0)

    @pl.when(i == 0)
    def _():
        acc_ref[0] = 0.0

    xb = x_ref[...]
    s = jnp.sum(jnp.exp(xb), axis=0)
    acc_ref[0] += jnp.sum(jnp.log(s))

    @pl.when(i == NBT - 1)
    def _():
        out_ref[...] = jnp.broadcast_to(acc_ref[0], (1, 1))


def _combine_body(lse_ref, part_ref, out_ref):
    val = (lse_ref[0, 0] - jnp.sum(part_ref[...])) / N
    out_ref[...] = jnp.broadcast_to(val, (1, 1))


@functools.cache
def _sc_kernels():
    sc_params = pltpu.CompilerParams(needs_layout_passes=False)
    hist_k = pl.kernel(
        _hist_body,
        out_type=jax.ShapeDtypeStruct((NC * HIST,), jnp.int32),
        mesh=_mesh(),
        scratch_types=[
            [pltpu.VMEM((CH,), jnp.int32) for _ in range(2)],
            [pltpu.VMEM((CH,), jnp.int32) for _ in range(2)],
            [pltpu.VMEM((128,), jnp.int32) for _ in range(CH // 128)],
            pltpu.VMEM((128,), jnp.int32),
            pltpu.VMEM((HIST // NS,), jnp.int32),
            pltpu.VMEM_SHARED((HIST,), jnp.int32),
            pltpu.SemaphoreType.DMA,
            [pltpu.SemaphoreType.DMA for _ in range(2)],
        ],
        compiler_params=sc_params,
    )
    argmax_k = pl.kernel(
        _argmax_body,
        out_type=jax.ShapeDtypeStruct((NSEG,), jnp.int32),
        mesh=_mesh(),
        scratch_types=[
            pltpu.VMEM((NSEG // NW * C,), jnp.int32),
            pltpu.VMEM((NSEG // NW * C,), jnp.int32),
            pltpu.VMEM((NSEG // NW,), jnp.int32),
            pltpu.SemaphoreType.DMA,
        ],
        compiler_params=sc_params,
    )
    pick_k = pl.kernel(
        _pick_body,
        out_type=jax.ShapeDtypeStruct((NW * L * 2,), jnp.float32),
        mesh=_mesh(),
        scratch_types=[
            pltpu.VMEM((NSEG,), jnp.int32),
            [pltpu.VMEM((CH,), jnp.int32) for _ in range(2)],
            pltpu.VMEM((CH,), jnp.int32),
            pltpu.VMEM((CH,), jnp.float32),
            pltpu.VMEM((L,), jnp.float32),
            pltpu.SemaphoreType.DMA,
            [pltpu.SemaphoreType.DMA for _ in range(2)],
        ],
        compiler_params=sc_params,
    )
    return hist_k, argmax_k, pick_k


def kernel(output, target, segment_ids):
    hist_k, argmax_k, pick_k = _sc_kernels()
    seg = segment_ids.astype(jnp.int32)
    tgt = target.astype(jnp.int32)
    xt = output.T
    flat = (xt.reshape(8, 8, N // 128, 128)
              .transpose(0, 2, 1, 3).reshape(-1))
    sum_lse = pl.pallas_call(
        _lse_body,
        grid=(NBT,),
        in_specs=[pl.BlockSpec((C, BLKC), lambda i: (0, i))],
        out_specs=pl.BlockSpec((1, 1), lambda i: (0, 0)),
        out_shape=jax.ShapeDtypeStruct((1, 1), jnp.float32),
        scratch_shapes=[pltpu.SMEM((1,), jnp.float32)],
    )(xt)
    hists = hist_k(seg, tgt)
    majority = argmax_k(hists)
    partials = pick_k(flat, seg, majority)
    loss = pl.pallas_call(
        _combine_body,
        in_specs=[
            pl.BlockSpec((1, 1), lambda: (0, 0)),
            pl.BlockSpec((8, 128), lambda: (0, 0)),
        ],
        out_specs=pl.BlockSpec((1, 1), lambda: (0, 0)),
        out_shape=jax.ShapeDtypeStruct((1, 1), jnp.float32),
    )(sum_lse, partials.reshape(8, 128))
    return loss.reshape(())
